# Initial kernel scaffold; baseline (speedup 1.0000x reference)
import jax
import jax.numpy as jnp
from jax.experimental import pallas as pl


def kernel(x, edge_index, pos, W_in, b_in, Wl, bl, Wr, gamma, beta, rm, rv, W_out, b_out):
    raise NotImplementedError("write your pallas kernel here")


# speedup vs baseline: 3.3651x; 3.3651x over previous
import functools
import math

import jax
import jax.numpy as jnp
from jax import lax
from jax.experimental import pallas as pl
from jax.experimental.pallas import tpu as pltpu
from jax.experimental.pallas import tpu_sc as plsc

N = 10000
E = 320000
H = 128
PE = 32
L = 3

NC = 2
NS = 16
NW = NC * NS
CH = 128
EPW = 10112
E_PAD = NW * EPW
NCH = EPW // CH
N_ACC = 10240
RPT = N_ACC // NS
BM = 2000

_HIGH = jax.lax.Precision.HIGHEST


def _dot(a, b):
    return jnp.dot(a, b, precision=_HIGH, preferred_element_type=jnp.float32)


def _sc_agg_body(h_hbm, src_hbm, dst_hbm, part_hbm, degp_hbm,
                 idx_s, idx_d, rows, ones_v, zrow, acc, dacc, sem):
    cid = lax.axis_index("c")
    sid = lax.axis_index("s")
    wid = sid * NC + cid
    t0 = sid * RPT

    def fill(k, _):
        ones_v[pl.ds(k * 16, 16)] = jnp.full((16,), 1.0, jnp.float32)
        zrow[pl.ds(k * 16, 16)] = jnp.zeros((16,), jnp.float32)
        return 0
    lax.fori_loop(0, CH // 16, fill, 0)

    def zrows(k, _):
        r = k // (H // 16)
        c = k % (H // 16)
        rows[r, pl.ds(c * 16, 16)] = jnp.zeros((16,), jnp.float32)
        return 0
    lax.fori_loop(0, CH * (H // 16), zrows, 0)

    def zacc(k, _):
        pltpu.sync_copy(rows, acc.at[pl.ds(t0 + k * CH, CH)])
        pltpu.sync_copy(zrow, dacc.at[pl.ds(t0 + k * CH, CH)])
        return 0
    lax.fori_loop(0, RPT // CH, zacc, 0)

    plsc.subcore_barrier()

    def chunk(j, _):
        base = wid * EPW + j * CH
        pltpu.sync_copy(src_hbm.at[pl.ds(base, CH)], idx_s)
        pltpu.sync_copy(dst_hbm.at[pl.ds(base, CH)], idx_d)
        pltpu.async_copy(h_hbm.at[idx_s], rows, sem).wait()
        pltpu.sync_copy(rows, acc.at[idx_d], add=True)
        pltpu.sync_copy(ones_v, dacc.at[idx_d], add=True)
        return 0
    lax.fori_loop(0, NCH, chunk, 0)

    plsc.subcore_barrier()

    def wout(k, _):
        r = t0 + k * CH
        pltpu.sync_copy(acc.at[pl.ds(r, CH)], part_hbm.at[cid, pl.ds(r, CH)])
        return 0
    lax.fori_loop(0, RPT // CH, wout, 0)
    pltpu.sync_copy(dacc.at[pl.ds(t0, RPT)], degp_hbm.at[cid, pl.ds(t0, RPT)])


@functools.partial(jax.jit, static_argnames=())
def _sc_agg(h, src_pad, dst_pad):
    mesh = plsc.VectorSubcoreMesh(core_axis_name="c", subcore_axis_name="s",
                                  num_cores=NC, num_subcores=NS)
    kern = pl.kernel(
        _sc_agg_body,
        out_type=(
            jax.ShapeDtypeStruct((NC, N_ACC, H), jnp.float32),
            jax.ShapeDtypeStruct((NC, N_ACC), jnp.float32),
        ),
        mesh=mesh,
        scratch_types=[
            pltpu.VMEM((CH,), jnp.int32),
            pltpu.VMEM((CH,), jnp.int32),
            pltpu.VMEM((CH, H), jnp.float32),
            pltpu.VMEM((CH,), jnp.float32),
            pltpu.VMEM((CH,), jnp.float32),
            pltpu.VMEM_SHARED((N_ACC, H), jnp.float32),
            pltpu.VMEM_SHARED((N_ACC,), jnp.float32),
            pltpu.SemaphoreType.DMA,
        ],
    )
    return kern(h, src_pad, dst_pad)


def _tc_in_body(x_ref, pos_ref, wxt_ref, wpet_ref, b_ref, fr_ref, out_ref):
    pes = []
    for i in range(2):
        ang = pos_ref[:, i:i + 1] * fr_ref[...]
        pes.append(jnp.concatenate([jnp.sin(ang), jnp.cos(ang)], axis=1))
    pe = jnp.concatenate(pes, axis=1)
    h = _dot(x_ref[...], wxt_ref[...]) + _dot(pe, wpet_ref[...]) + b_ref[...]
    out_ref[...] = jnp.maximum(h, 0.0)


def _tc_in(x, pos, wxt, wpet, b_in, freqs):
    grid = (N // BM,)
    return pl.pallas_call(
        _tc_in_body,
        grid=grid,
        in_specs=[
            pl.BlockSpec((BM, H), lambda i: (i, 0)),
            pl.BlockSpec((BM, 2), lambda i: (i, 0)),
            pl.BlockSpec((H, H), lambda i: (0, 0)),
            pl.BlockSpec((PE, H), lambda i: (0, 0)),
            pl.BlockSpec((1, H), lambda i: (0, 0)),
            pl.BlockSpec((1, PE // 4), lambda i: (0, 0)),
        ],
        out_specs=pl.BlockSpec((BM, H), lambda i: (i, 0)),
        out_shape=jax.ShapeDtypeStruct((N, H), jnp.float32),
    )(x, pos, wxt, wpet, b_in, freqs)


def _layer_block(part_ref, degp_ref, h_ref, wlt_ref, wrt_ref, bl_ref,
                 sc_ref, sh_ref):
    p = part_ref[0] + part_ref[1]
    d = degp_ref[0] + degp_ref[1]
    agg = p * (1.0 / jnp.maximum(d, 1.0))
    s = _dot(agg, wlt_ref[...]) + _dot(h_ref[...], wrt_ref[...]) + bl_ref[...]
    s = s * sc_ref[...] + sh_ref[...]
    return jnp.maximum(s, 0.0) + h_ref[...]


def _tc_layer_body(part_ref, degp_ref, h_ref, wlt_ref, wrt_ref, bl_ref,
                   sc_ref, sh_ref, out_ref):
    out_ref[...] = _layer_block(part_ref, degp_ref, h_ref, wlt_ref, wrt_ref,
                                bl_ref, sc_ref, sh_ref)


def _layer_specs():
    return [
        pl.BlockSpec((NC, BM, H), lambda i: (0, i, 0)),
        pl.BlockSpec((NC, BM, 1), lambda i: (0, i, 0)),
        pl.BlockSpec((BM, H), lambda i: (i, 0)),
        pl.BlockSpec((H, H), lambda i: (0, 0)),
        pl.BlockSpec((H, H), lambda i: (0, 0)),
        pl.BlockSpec((1, H), lambda i: (0, 0)),
        pl.BlockSpec((1, H), lambda i: (0, 0)),
        pl.BlockSpec((1, H), lambda i: (0, 0)),
    ]


def _tc_layer(part, degp, h, wlt, wrt, bl, bnsc, bnsh):
    grid = (N // BM,)
    return pl.pallas_call(
        _tc_layer_body,
        grid=grid,
        in_specs=_layer_specs(),
        out_specs=pl.BlockSpec((BM, H), lambda i: (i, 0)),
        out_shape=jax.ShapeDtypeStruct((N, H), jnp.float32),
    )(part, degp, h, wlt, wrt, bl, bnsc, bnsh)


def _tc_final_body(part_ref, degp_ref, h_ref, wlt_ref, wrt_ref, bl_ref,
                   sc_ref, sh_ref, wot_ref, bo_ref, out_ref, acc_ref):
    i = pl.program_id(0)

    @pl.when(i == 0)
    def _():
        acc_ref[...] = jnp.zeros_like(acc_ref)

    h3 = _layer_block(part_ref, degp_ref, h_ref, wlt_ref, wrt_ref, bl_ref,
                      sc_ref, sh_ref)
    acc_ref[...] += jnp.sum(h3, axis=0, keepdims=True)

    @pl.when(i == pl.num_programs(0) - 1)
    def _():
        m = acc_ref[...] * (1.0 / N)
        out_ref[...] = _dot(m, wot_ref[...]) + bo_ref[...]


def _tc_final(part, degp, h, wlt, wrt, bl, bnsc, bnsh, wot, b_out):
    grid = (N // BM,)
    return pl.pallas_call(
        _tc_final_body,
        grid=grid,
        in_specs=_layer_specs() + [
            pl.BlockSpec((H, H), lambda i: (0, 0)),
            pl.BlockSpec((1, H), lambda i: (0, 0)),
        ],
        out_specs=pl.BlockSpec((1, H), lambda i: (0, 0)),
        out_shape=jax.ShapeDtypeStruct((1, H), jnp.float32),
        scratch_shapes=[pltpu.VMEM((1, H), jnp.float32)],
        compiler_params=pltpu.CompilerParams(
            dimension_semantics=("arbitrary",)),
    )(part, degp, h, wlt, wrt, bl, bnsc, bnsh, wot, b_out)


def kernel(x, edge_index, pos, W_in, b_in, Wl, bl, Wr, gamma, beta, rm, rv,
           W_out, b_out):
    src = edge_index[0]
    dst = edge_index[1]
    pad = E_PAD - E
    src_pad = jnp.concatenate([src, jnp.zeros((pad,), jnp.int32)])
    dst_pad = jnp.concatenate([dst, jnp.full((pad,), N, jnp.int32)])

    perm = []
    for i in range(2):
        perm += [i * 16 + 2 * k for k in range(8)]
        perm += [i * 16 + 2 * k + 1 for k in range(8)]
    wxt = W_in[:, :128].T
    wpet = W_in[:, 128:][:, jnp.array(perm)].T
    freqs = jnp.linspace(1.0, 10.0, PE // 4).reshape(1, -1)

    bn_scale = gamma / jnp.sqrt(rv + 1e-5)
    bn_shift = beta - rm * bn_scale

    h = _tc_in(x, pos, wxt, wpet, b_in.reshape(1, H), freqs)

    for i in range(L):
        part, degp = _sc_agg(h, src_pad, dst_pad)
        degp = degp.reshape(NC, N_ACC, 1)
        args = (part, degp, h, Wl[i].T, Wr[i].T, bl[i].reshape(1, H),
                bn_scale[i].reshape(1, H), bn_shift[i].reshape(1, H))
        if i < L - 1:
            h = _tc_layer(*args)
        else:
            out = _tc_final(*args, W_out.T, b_out.reshape(1, H))
    return out
